# baseline (speedup 1.0000x reference)
import functools

import jax
import jax.numpy as jnp
from jax import lax
from jax.experimental import pallas as pl
from jax.experimental.pallas import tpu as pltpu
from jax.experimental.pallas import tpu_sc as plsc

N = 10000
E = 320000
H = 3
F = 64
HF = H * F
KHOP = 3
NEG = 0.2

NC = 2
NS = 16
NW = NC * NS
EPT = E // NW
CH = 80
NCHUNK = EPT // CH
CH2 = 2000
RPT = N // NS

_mesh = plsc.VectorSubcoreMesh(core_axis_name="c", subcore_axis_name="s")
_sc_params = pltpu.CompilerParams(needs_layout_passes=False,
                                  use_tc_tiling_on_sc=False)


def _proj_body(feat_ref, wt_ref, al_ref, ar_ref, ft_ref, el_ref, er_ref):
    ft = jnp.dot(feat_ref[...], wt_ref[...], preferred_element_type=jnp.float32)
    ft_ref[...] = ft
    el_ref[...] = jnp.dot(ft, al_ref[...], preferred_element_type=jnp.float32)
    er_ref[...] = jnp.dot(ft, ar_ref[...], preferred_element_type=jnp.float32)


def _proj_call(feat, wt, al, ar):
    return pl.pallas_call(
        _proj_body,
        out_shape=[
            jax.ShapeDtypeStruct((N, HF), jnp.float32),
            jax.ShapeDtypeStruct((N, H), jnp.float32),
            jax.ShapeDtypeStruct((N, H), jnp.float32),
        ],
    )(feat, wt, al, ar)


@functools.partial(
    pl.kernel,
    out_type=jax.ShapeDtypeStruct((NW * N * 4,), jnp.float32),
    mesh=_mesh,
    compiler_params=_sc_params,
    scratch_types=[
        pltpu.VMEM((N * H,), jnp.float32),
        pltpu.VMEM((N * H,), jnp.float32),
        pltpu.VMEM((N * 4,), jnp.float32),
        pltpu.VMEM((EPT,), jnp.int32),
        pltpu.VMEM((EPT,), jnp.int32),
    ],
)
def _stats_kernel(src_hbm, dst_hbm, el_hbm, er_hbm, out_hbm,
                  el_v, er_v, part_v, src_v, dst_v):
    c = lax.axis_index("c")
    s = lax.axis_index("s")
    wid = c * NS + s
    pltpu.sync_copy(el_hbm, el_v)
    pltpu.sync_copy(er_hbm, er_v)
    base_e = wid * EPT
    pltpu.sync_copy(src_hbm.at[pl.ds(base_e, EPT)], src_v)
    pltpu.sync_copy(dst_hbm.at[pl.ds(base_e, EPT)], dst_v)

    zero16 = jnp.zeros((16,), jnp.float32)

    def zbody(i, _):
        part_v[pl.ds(i * 16, 16)] = zero16
        return 0

    lax.fori_loop(0, N * 4 // 16, zbody, 0)

    ones16 = jnp.ones((16,), jnp.float32)

    def vec_body(v, _):
        s16 = src_v[pl.ds(v * 16, 16)]
        d16 = dst_v[pl.ds(v * 16, 16)]
        s3 = s16 * H
        d3 = d16 * H
        d4 = d16 * 4
        for h in range(H):
            elv = plsc.load_gather(el_v, [s3 + h])
            erv = plsc.load_gather(er_v, [d3 + h])
            e = elv + erv
            e = jnp.where(e >= 0.0, e, e * NEG)
            plsc.addupdate_scatter(part_v, [d4 + h], jnp.exp(e))
        plsc.addupdate_scatter(part_v, [d4 + 3], ones16)
        return 0

    lax.fori_loop(0, EPT // 16, vec_body, 0, unroll=2)
    pltpu.sync_copy(part_v, out_hbm.at[pl.ds(wid * N * 4, N * 4)])


def _mid_body(sp_ref, mid_ref):
    stats = jnp.sum(sp_ref[...], axis=0)
    col = lax.rem(lax.broadcasted_iota(jnp.int32, (N * 4,), 0), 4)
    ise = 1.0 / jnp.maximum(stats, 1e-16)
    dn = lax.rsqrt(jnp.maximum(stats, 1.0))
    mid_ref[...] = jnp.where(col < 3, ise, dn)


def _mid_call(spart):
    return pl.pallas_call(
        _mid_body,
        out_shape=jax.ShapeDtypeStruct((N * 4,), jnp.float32),
    )(spart)


@functools.partial(
    pl.kernel,
    out_type=jax.ShapeDtypeStruct((H * E,), jnp.float32),
    mesh=_mesh,
    compiler_params=_sc_params,
    scratch_types=[
        pltpu.VMEM((N * H,), jnp.float32),
        pltpu.VMEM((N * H,), jnp.float32),
        pltpu.VMEM((N * 4,), jnp.float32),
        pltpu.VMEM((CH2,), jnp.int32),
        pltpu.VMEM((CH2,), jnp.int32),
        pltpu.VMEM((H, CH2), jnp.float32),
    ],
)
def _edge_kernel(src_hbm, dst_hbm, el_hbm, er_hbm, mid_hbm, out_hbm,
                 el_v, er_v, mid_v, src_v, dst_v, a_v):
    c = lax.axis_index("c")
    s = lax.axis_index("s")
    wid = c * NS + s
    pltpu.sync_copy(el_hbm, el_v)
    pltpu.sync_copy(er_hbm, er_v)
    pltpu.sync_copy(mid_hbm, mid_v)

    base_e = wid * EPT

    def chunk_body(ci, _):
        off = base_e + ci * CH2
        pltpu.sync_copy(src_hbm.at[pl.ds(off, CH2)], src_v)
        pltpu.sync_copy(dst_hbm.at[pl.ds(off, CH2)], dst_v)

        def vec_body(v, _):
            sl = pl.ds(v * 16, 16)
            s16 = src_v[sl]
            d16 = dst_v[sl]
            s3 = s16 * H
            d3 = d16 * H
            s4 = s16 * 4
            d4 = d16 * 4
            dns = plsc.load_gather(mid_v, [s4 + 3])
            dnd = plsc.load_gather(mid_v, [d4 + 3])
            dd = dns * dnd
            for h in range(H):
                elv = plsc.load_gather(el_v, [s3 + h])
                erv = plsc.load_gather(er_v, [d3 + h])
                e = elv + erv
                e = jnp.where(e >= 0.0, e, e * NEG)
                ise = plsc.load_gather(mid_v, [d4 + h])
                a_v[h, sl] = jnp.exp(e) * ise * dd
            return 0

        lax.fori_loop(0, CH2 // 16, vec_body, 0, unroll=2)
        for h in range(H):
            pltpu.sync_copy(a_v.at[h], out_hbm.at[pl.ds(h * E + off, CH2)])
        return 0

    lax.fori_loop(0, EPT // CH2, chunk_body, 0)


NBUF = 6


@functools.partial(
    pl.kernel,
    out_type=jax.ShapeDtypeStruct((NC * H * N, F), jnp.float32),
    mesh=_mesh,
    compiler_params=_sc_params,
    scratch_types=[
        pltpu.VMEM_SHARED((N, F), jnp.float32),
        pltpu.VMEM((EPT,), jnp.int32),
        pltpu.VMEM((NCHUNK, CH), jnp.int32),
        pltpu.VMEM((EPT,), jnp.int32),
        pltpu.VMEM((EPT,), jnp.float32),
        [pltpu.VMEM((CH, F), jnp.float32) for _ in range(NBUF)],
        [pltpu.SemaphoreType.DMA for _ in range(NBUF)],
        [pltpu.SemaphoreType.DMA for _ in range(NBUF)],
    ],
)
def _hop_kernel(src_hbm, dst3_hbm, a_hbm, hin3_hbm, zrow_hbm, out_hbm,
                acc_sh, src_v, dst_v, gi_v, a_v, rows, gsems, ssems):
    c = lax.axis_index("c")
    s = lax.axis_index("s")
    wid = c * NS + s
    base_e = wid * EPT
    pltpu.sync_copy(src_hbm.at[pl.ds(base_e, EPT)], src_v)
    pltpu.sync_copy(dst3_hbm.at[wid], dst_v)
    def fill_and_gather(b, ci, h):
        pltpu.async_copy(hin3_hbm.at[gi_v.at[pl.ds(ci * CH, CH)]], rows[b],
                         gsems[b])

    def scale_rows(b, ci, h):
        rv = rows[b]

        def rbody(r, _):
            a_s = plsc.load_gather(a_v, [jnp.full((16,), ci * CH + r,
                                                  jnp.int32)])
            for q in range(F // 16):
                slc = pl.ds(q * 16, 16)
                rv[r, slc] = rv[r, slc] * a_s
            return 0

        lax.fori_loop(0, CH, rbody, 0, unroll=4)

    def process(b, cc, h):
        nxt = cc + (NBUF - 1)
        bp = (b + NBUF - 1) % NBUF

        @pl.when(nxt < NCHUNK)
        def _():
            @pl.when(cc > 0)
            def _():
                pltpu.make_async_copy(rows[bp], acc_sh.at[dst_v.at[0]],
                                      ssems[bp]).wait()
            fill_and_gather(bp, nxt, h)

        pltpu.make_async_copy(hin3_hbm.at[gi_v.at[pl.ds(cc * CH, CH)]],
                              rows[b], gsems[b]).wait()
        scale_rows(b, cc, h)
        pltpu.async_copy(rows[b], acc_sh.at[dst_v.at[cc]], ssems[b], add=True)

    for h in range(H):
        pltpu.sync_copy(zrow_hbm, acc_sh.at[pl.ds(s * RPT, RPT), :])
        pltpu.sync_copy(a_hbm.at[pl.ds(h * E + base_e, EPT)], a_v)

        hn = h * N

        def gfill(k, _):
            sl = pl.ds(k * 16, 16)
            gi_v[sl] = src_v[sl] + hn
            return 0

        lax.fori_loop(0, EPT // 16, gfill, 0, unroll=8)
        plsc.subcore_barrier()

        for b in range(NBUF - 1):
            fill_and_gather(b, b, h)

        def quad_body(p, _):
            ci = p * NBUF
            for b in range(NBUF):
                process(b, ci + b, h)
            return 0

        lax.fori_loop(0, NCHUNK // NBUF, quad_body, 0)
        for b in range(NCHUNK % NBUF):
            process(b, (NCHUNK // NBUF) * NBUF + b, h)

        for b in range(NBUF):
            pltpu.make_async_copy(rows[b], acc_sh.at[dst_v.at[0]],
                                  ssems[b]).wait()
        plsc.subcore_barrier()
        pltpu.sync_copy(acc_sh.at[pl.ds(s * RPT, RPT), :],
                        out_hbm.at[pl.ds(c * H * N + h * N + s * RPT, RPT), :])
        plsc.subcore_barrier()


def _add_body(hp_ref, out3_ref, outf_ref):
    p = hp_ref[0] + hp_ref[1]
    out3_ref[...] = p
    for h in range(H):
        outf_ref[:, h * F:(h + 1) * F] = p[h * N:(h + 1) * N, :]


def _add_call(hp):
    return pl.pallas_call(
        _add_body,
        out_shape=[
            jax.ShapeDtypeStruct((H * N, F), jnp.float32),
            jax.ShapeDtypeStruct((N, HF), jnp.float32),
        ],
    )(hp)


_BN = 2000


def _final_body(ft_ref, h1_ref, h2_ref, h3_ref, pe_ref, hl_ref, hr_ref,
                e3_ref, out_ref):
    pe = pe_ref[...]
    g = [
        ft_ref[...] + pe[0:1, :],
        h1_ref[...] + pe[1:2, :],
        h2_ref[...] + pe[2:3, :],
        h3_ref[...] + pe[3:4, :],
    ]
    hl = hl_ref[...]
    hr = hr_ref[...]
    al_ = jnp.dot(g[0], hl, preferred_element_type=jnp.float32)
    ah = [None] * (KHOP + 1)
    for k in range(KHOP + 1):
        rk = jnp.dot(g[k], hr, preferred_element_type=jnp.float32)
        x = rk + al_
        ah[k] = jnp.where(x >= 0.0, x, x * NEG)
    m = jnp.maximum(jnp.maximum(ah[0], ah[1]), jnp.maximum(ah[2], ah[3]))
    ex = [jnp.exp(a - m) for a in ah]
    ssum = ex[0] + ex[1] + ex[2] + ex[3]
    e3 = e3_ref[...]
    out = jnp.zeros_like(g[0])
    for k in range(KHOP + 1):
        w = ex[k] / ssum
        out = out + g[k] * jnp.dot(w, e3, preferred_element_type=jnp.float32)
    out_ref[...] = out


def _final_call(ft, h1, h2, h3, pe, hl, hr, e3):
    blk = lambda i: (i, 0)
    full2 = pl.BlockSpec((_BN, HF), blk)
    return pl.pallas_call(
        _final_body,
        grid=(N // _BN,),
        in_specs=[
            full2, full2, full2, full2,
            pl.BlockSpec((KHOP + 1, HF), lambda i: (0, 0)),
            pl.BlockSpec((HF, H), lambda i: (0, 0)),
            pl.BlockSpec((HF, H), lambda i: (0, 0)),
            pl.BlockSpec((H, HF), lambda i: (0, 0)),
        ],
        out_specs=full2,
        out_shape=jax.ShapeDtypeStruct((N, HF), jnp.float32),
    )(ft, h1, h2, h3, pe, hl, hr, e3)


def _blockdiag(vec):
    eye = jnp.eye(H, dtype=jnp.float32)
    return (vec[:, :, None] * eye[:, None, :]).reshape(HF, H)


def kernel(feat, edge_index, W_fc, attn_l, attn_r, position_emb,
           hop_attn_l, hop_attn_r):
    src = edge_index[0].astype(jnp.int32)
    dst = edge_index[1].astype(jnp.int32)
    dst3 = dst.reshape(NW, NCHUNK, CH)
    wt = W_fc.T
    al = _blockdiag(attn_l[0])
    ar = _blockdiag(attn_r[0])
    hl = _blockdiag(hop_attn_l[0])
    hr = _blockdiag(hop_attn_r[0])
    e3 = jnp.repeat(jnp.eye(H, dtype=jnp.float32), F, axis=1)
    pe = position_emb.reshape(KHOP + 1, HF)
    zrow = jnp.zeros((RPT, F), jnp.float32)

    ft, el, er = _proj_call(feat, wt, al, ar)
    elf = el.reshape(-1)
    erf = er.reshape(-1)
    spart = _stats_kernel(src, dst, elf, erf).reshape(NW, N * 4)
    mid = _mid_call(spart)
    a4 = _edge_kernel(src, dst, elf, erf, mid)

    ft3 = ft.reshape(N, H, F).transpose(1, 0, 2).reshape(H * N, F)

    h3 = ft3
    hsf = [ft]
    for _ in range(KHOP):
        hp = _hop_kernel(src, dst3, a4, h3, zrow).reshape(NC, H * N, F)
        h3, hflat = _add_call(hp)
        hsf.append(hflat)
    rst = _final_call(hsf[0], hsf[1], hsf[2], hsf[3], pe, hl, hr, e3)
    return rst.reshape(N, H, F)


# speedup vs baseline: 1.2017x; 1.2017x over previous
import functools

import jax
import jax.numpy as jnp
from jax import lax
from jax.experimental import pallas as pl
from jax.experimental.pallas import tpu as pltpu
from jax.experimental.pallas import tpu_sc as plsc

N = 10000
E = 320000
H = 3
F = 64
HF = H * F
KHOP = 3
NEG = 0.2

NC = 2
NS = 16
NW = NC * NS
EPT = E // NW
CH = 80
NCHUNK = EPT // CH
CH2 = 2000
RPT = N // NS

_mesh = plsc.VectorSubcoreMesh(core_axis_name="c", subcore_axis_name="s")
_sc_params = pltpu.CompilerParams(needs_layout_passes=False,
                                  use_tc_tiling_on_sc=False)


def _proj_body(feat_ref, wt_ref, al_ref, ar_ref, ft_ref, el_ref, er_ref):
    ft = jnp.dot(feat_ref[...], wt_ref[...], preferred_element_type=jnp.float32)
    ft_ref[...] = ft
    el_ref[...] = jnp.dot(ft, al_ref[...], preferred_element_type=jnp.float32)
    er_ref[...] = jnp.dot(ft, ar_ref[...], preferred_element_type=jnp.float32)


def _proj_call(feat, wt, al, ar):
    return pl.pallas_call(
        _proj_body,
        out_shape=[
            jax.ShapeDtypeStruct((N, HF), jnp.float32),
            jax.ShapeDtypeStruct((N, H), jnp.float32),
            jax.ShapeDtypeStruct((N, H), jnp.float32),
        ],
    )(feat, wt, al, ar)


@functools.partial(
    pl.kernel,
    out_type=jax.ShapeDtypeStruct((NW * N * 4,), jnp.float32),
    mesh=_mesh,
    compiler_params=_sc_params,
    scratch_types=[
        pltpu.VMEM((N * H,), jnp.float32),
        pltpu.VMEM((N * H,), jnp.float32),
        pltpu.VMEM((N * 4,), jnp.float32),
        pltpu.VMEM((EPT,), jnp.int32),
        pltpu.VMEM((EPT,), jnp.int32),
    ],
)
def _stats_kernel(src_hbm, dst_hbm, el_hbm, er_hbm, out_hbm,
                  el_v, er_v, part_v, src_v, dst_v):
    c = lax.axis_index("c")
    s = lax.axis_index("s")
    wid = c * NS + s
    pltpu.sync_copy(el_hbm, el_v)
    pltpu.sync_copy(er_hbm, er_v)
    base_e = wid * EPT
    pltpu.sync_copy(src_hbm.at[pl.ds(base_e, EPT)], src_v)
    pltpu.sync_copy(dst_hbm.at[pl.ds(base_e, EPT)], dst_v)

    zero16 = jnp.zeros((16,), jnp.float32)

    def zbody(i, _):
        part_v[pl.ds(i * 16, 16)] = zero16
        return 0

    lax.fori_loop(0, N * 4 // 16, zbody, 0)

    ones16 = jnp.ones((16,), jnp.float32)

    def vec_body(v, _):
        s16 = src_v[pl.ds(v * 16, 16)]
        d16 = dst_v[pl.ds(v * 16, 16)]
        s3 = s16 * H
        d3 = d16 * H
        d4 = d16 * 4
        for h in range(H):
            elv = plsc.load_gather(el_v, [s3 + h])
            erv = plsc.load_gather(er_v, [d3 + h])
            e = elv + erv
            e = jnp.where(e >= 0.0, e, e * NEG)
            plsc.addupdate_scatter(part_v, [d4 + h], jnp.exp(e))
        plsc.addupdate_scatter(part_v, [d4 + 3], ones16)
        return 0

    lax.fori_loop(0, EPT // 16, vec_body, 0, unroll=2)
    pltpu.sync_copy(part_v, out_hbm.at[pl.ds(wid * N * 4, N * 4)])


def _mid_body(sp_ref, mid_ref):
    stats = jnp.sum(sp_ref[...], axis=0)
    col = lax.rem(lax.broadcasted_iota(jnp.int32, (N * 4,), 0), 4)
    ise = 1.0 / jnp.maximum(stats, 1e-16)
    dn = lax.rsqrt(jnp.maximum(stats, 1.0))
    mid_ref[...] = jnp.where(col < 3, ise, dn)


def _mid_call(spart):
    return pl.pallas_call(
        _mid_body,
        out_shape=jax.ShapeDtypeStruct((N * 4,), jnp.float32),
    )(spart)


@functools.partial(
    pl.kernel,
    out_type=jax.ShapeDtypeStruct((H * E,), jnp.float32),
    mesh=_mesh,
    compiler_params=_sc_params,
    scratch_types=[
        pltpu.VMEM((N * H,), jnp.float32),
        pltpu.VMEM((N * H,), jnp.float32),
        pltpu.VMEM((N * 4,), jnp.float32),
        pltpu.VMEM((CH2,), jnp.int32),
        pltpu.VMEM((CH2,), jnp.int32),
        pltpu.VMEM((H, CH2), jnp.float32),
    ],
)
def _edge_kernel(src_hbm, dst_hbm, el_hbm, er_hbm, mid_hbm, out_hbm,
                 el_v, er_v, mid_v, src_v, dst_v, a_v):
    c = lax.axis_index("c")
    s = lax.axis_index("s")
    wid = c * NS + s
    pltpu.sync_copy(el_hbm, el_v)
    pltpu.sync_copy(er_hbm, er_v)
    pltpu.sync_copy(mid_hbm, mid_v)

    base_e = wid * EPT

    def chunk_body(ci, _):
        off = base_e + ci * CH2
        pltpu.sync_copy(src_hbm.at[pl.ds(off, CH2)], src_v)
        pltpu.sync_copy(dst_hbm.at[pl.ds(off, CH2)], dst_v)

        def vec_body(v, _):
            sl = pl.ds(v * 16, 16)
            s16 = src_v[sl]
            d16 = dst_v[sl]
            s3 = s16 * H
            d3 = d16 * H
            s4 = s16 * 4
            d4 = d16 * 4
            dns = plsc.load_gather(mid_v, [s4 + 3])
            dnd = plsc.load_gather(mid_v, [d4 + 3])
            dd = dns * dnd
            for h in range(H):
                elv = plsc.load_gather(el_v, [s3 + h])
                erv = plsc.load_gather(er_v, [d3 + h])
                e = elv + erv
                e = jnp.where(e >= 0.0, e, e * NEG)
                ise = plsc.load_gather(mid_v, [d4 + h])
                a_v[h, sl] = jnp.exp(e) * ise * dd
            return 0

        lax.fori_loop(0, CH2 // 16, vec_body, 0, unroll=2)
        for h in range(H):
            pltpu.sync_copy(a_v.at[h], out_hbm.at[pl.ds(h * E + off, CH2)])
        return 0

    lax.fori_loop(0, EPT // CH2, chunk_body, 0)


NBUF = 6
PDIST = 3


@functools.partial(
    pl.kernel,
    out_type=jax.ShapeDtypeStruct((NC * H * N, F), jnp.float32),
    mesh=_mesh,
    compiler_params=_sc_params,
    scratch_types=[
        pltpu.VMEM_SHARED((N, F), jnp.float32),
        pltpu.VMEM((EPT,), jnp.int32),
        pltpu.VMEM((NCHUNK, CH), jnp.int32),
        pltpu.VMEM((EPT,), jnp.int32),
        pltpu.VMEM((EPT,), jnp.float32),
        [pltpu.VMEM((CH, F), jnp.float32) for _ in range(NBUF)],
        [pltpu.SemaphoreType.DMA for _ in range(NBUF)],
        [pltpu.SemaphoreType.DMA for _ in range(NBUF)],
    ],
)
def _hop_kernel(src_hbm, dst3_hbm, a_hbm, hin3_hbm, zrow_hbm, out_hbm,
                acc_sh, src_v, dst_v, gi_v, a_v, rows, gsems, ssems):
    c = lax.axis_index("c")
    s = lax.axis_index("s")
    wid = c * NS + s
    base_e = wid * EPT
    pltpu.sync_copy(src_hbm.at[pl.ds(base_e, EPT)], src_v)
    pltpu.sync_copy(dst3_hbm.at[wid], dst_v)
    def fill_and_gather(b, ci, h):
        pltpu.async_copy(hin3_hbm.at[gi_v.at[pl.ds(ci * CH, CH)]], rows[b],
                         gsems[b])

    def scale_rows(b, ci, h):
        rv = rows[b]

        def rbody(r, _):
            a_s = plsc.load_gather(a_v, [jnp.full((16,), ci * CH + r,
                                                  jnp.int32)])
            for q in range(F // 16):
                slc = pl.ds(q * 16, 16)
                rv[r, slc] = rv[r, slc] * a_s
            return 0

        lax.fori_loop(0, CH, rbody, 0, unroll=8)

    def process(b, cc, h):
        nxt = cc + PDIST
        bp = (b + PDIST) % NBUF

        @pl.when(nxt < NCHUNK)
        def _():
            @pl.when(cc >= NBUF - PDIST)
            def _():
                pltpu.make_async_copy(rows[bp], acc_sh.at[dst_v.at[0]],
                                      ssems[bp]).wait()
            fill_and_gather(bp, nxt, h)

        pltpu.make_async_copy(hin3_hbm.at[gi_v.at[pl.ds(cc * CH, CH)]],
                              rows[b], gsems[b]).wait()
        scale_rows(b, cc, h)
        pltpu.async_copy(rows[b], acc_sh.at[dst_v.at[cc]], ssems[b], add=True)

    for h in range(H):
        pltpu.sync_copy(zrow_hbm, acc_sh.at[pl.ds(s * RPT, RPT), :])
        pltpu.sync_copy(a_hbm.at[pl.ds(h * E + base_e, EPT)], a_v)

        hn = h * N

        def gfill(k, _):
            sl = pl.ds(k * 16, 16)
            gi_v[sl] = src_v[sl] + hn
            return 0

        lax.fori_loop(0, EPT // 16, gfill, 0, unroll=8)
        plsc.subcore_barrier()

        for b in range(PDIST):
            fill_and_gather(b, b, h)

        def quad_body(p, _):
            ci = p * NBUF
            for b in range(NBUF):
                process(b, ci + b, h)
            return 0

        lax.fori_loop(0, NCHUNK // NBUF, quad_body, 0)
        for b in range(NCHUNK % NBUF):
            process(b, (NCHUNK // NBUF) * NBUF + b, h)

        for b in range(NBUF):
            pltpu.make_async_copy(rows[b], acc_sh.at[dst_v.at[0]],
                                  ssems[b]).wait()
        plsc.subcore_barrier()
        pltpu.sync_copy(acc_sh.at[pl.ds(s * RPT, RPT), :],
                        out_hbm.at[pl.ds(c * H * N + h * N + s * RPT, RPT), :])
        plsc.subcore_barrier()


def _add_body(hp_ref, out3_ref, outf_ref):
    p = hp_ref[0] + hp_ref[1]
    out3_ref[...] = p
    for h in range(H):
        outf_ref[:, h * F:(h + 1) * F] = p[h * N:(h + 1) * N, :]


def _add_call(hp):
    return pl.pallas_call(
        _add_body,
        out_shape=[
            jax.ShapeDtypeStruct((H * N, F), jnp.float32),
            jax.ShapeDtypeStruct((N, HF), jnp.float32),
        ],
    )(hp)


_BN = 2000


def _final_body(ft_ref, h1_ref, h2_ref, h3_ref, pe_ref, hl_ref, hr_ref,
                e3_ref, out_ref):
    pe = pe_ref[...]
    g = [
        ft_ref[...] + pe[0:1, :],
        h1_ref[...] + pe[1:2, :],
        h2_ref[...] + pe[2:3, :],
        h3_ref[...] + pe[3:4, :],
    ]
    hl = hl_ref[...]
    hr = hr_ref[...]
    al_ = jnp.dot(g[0], hl, preferred_element_type=jnp.float32)
    ah = [None] * (KHOP + 1)
    for k in range(KHOP + 1):
        rk = jnp.dot(g[k], hr, preferred_element_type=jnp.float32)
        x = rk + al_
        ah[k] = jnp.where(x >= 0.0, x, x * NEG)
    m = jnp.maximum(jnp.maximum(ah[0], ah[1]), jnp.maximum(ah[2], ah[3]))
    ex = [jnp.exp(a - m) for a in ah]
    ssum = ex[0] + ex[1] + ex[2] + ex[3]
    e3 = e3_ref[...]
    out = jnp.zeros_like(g[0])
    for k in range(KHOP + 1):
        w = ex[k] / ssum
        out = out + g[k] * jnp.dot(w, e3, preferred_element_type=jnp.float32)
    out_ref[...] = out


def _final_call(ft, h1, h2, h3, pe, hl, hr, e3):
    blk = lambda i: (i, 0)
    full2 = pl.BlockSpec((_BN, HF), blk)
    return pl.pallas_call(
        _final_body,
        grid=(N // _BN,),
        in_specs=[
            full2, full2, full2, full2,
            pl.BlockSpec((KHOP + 1, HF), lambda i: (0, 0)),
            pl.BlockSpec((HF, H), lambda i: (0, 0)),
            pl.BlockSpec((HF, H), lambda i: (0, 0)),
            pl.BlockSpec((H, HF), lambda i: (0, 0)),
        ],
        out_specs=full2,
        out_shape=jax.ShapeDtypeStruct((N, HF), jnp.float32),
    )(ft, h1, h2, h3, pe, hl, hr, e3)


def _blockdiag(vec):
    eye = jnp.eye(H, dtype=jnp.float32)
    return (vec[:, :, None] * eye[:, None, :]).reshape(HF, H)


def kernel(feat, edge_index, W_fc, attn_l, attn_r, position_emb,
           hop_attn_l, hop_attn_r):
    src = edge_index[0].astype(jnp.int32)
    dst = edge_index[1].astype(jnp.int32)
    dst3 = dst.reshape(NW, NCHUNK, CH)
    wt = W_fc.T
    al = _blockdiag(attn_l[0])
    ar = _blockdiag(attn_r[0])
    hl = _blockdiag(hop_attn_l[0])
    hr = _blockdiag(hop_attn_r[0])
    e3 = jnp.repeat(jnp.eye(H, dtype=jnp.float32), F, axis=1)
    pe = position_emb.reshape(KHOP + 1, HF)
    zrow = jnp.zeros((RPT, F), jnp.float32)

    ft, el, er = _proj_call(feat, wt, al, ar)
    elf = el.reshape(-1)
    erf = er.reshape(-1)
    spart = _stats_kernel(src, dst, elf, erf).reshape(NW, N * 4)
    mid = _mid_call(spart)
    a4 = _edge_kernel(src, dst, elf, erf, mid)

    ft3 = ft.reshape(N, H, F).transpose(1, 0, 2).reshape(H * N, F)

    h3 = ft3
    hsf = [ft]
    for _ in range(KHOP):
        hp = _hop_kernel(src, dst3, a4, h3, zrow).reshape(NC, H * N, F)
        h3, hflat = _add_call(hp)
        hsf.append(hflat)
    rst = _final_call(hsf[0], hsf[1], hsf[2], hsf[3], pe, hl, hr, e3)
    return rst.reshape(N, H, F)


# speedup vs baseline: 1.6394x; 1.3642x over previous
import functools

import jax
import jax.numpy as jnp
from jax import lax
from jax.experimental import pallas as pl
from jax.experimental.pallas import tpu as pltpu
from jax.experimental.pallas import tpu_sc as plsc

N = 10000
E = 320000
H = 3
F = 64
HF = H * F
KHOP = 3
NEG = 0.2

NC = 2
NS = 16
NW = NC * NS
EPT = E // NW
CH = 80
NCHUNK = EPT // CH
CH2 = 2000
RPT = N // NS

_mesh = plsc.VectorSubcoreMesh(core_axis_name="c", subcore_axis_name="s")
_sc_params = pltpu.CompilerParams(needs_layout_passes=False,
                                  use_tc_tiling_on_sc=False)


def _proj_body(feat_ref, wt_ref, al_ref, ar_ref, ft_ref, el_ref, er_ref):
    ft = jnp.dot(feat_ref[...], wt_ref[...], preferred_element_type=jnp.float32)
    ft_ref[...] = ft
    el_ref[...] = jnp.dot(ft, al_ref[...], preferred_element_type=jnp.float32)
    er_ref[...] = jnp.dot(ft, ar_ref[...], preferred_element_type=jnp.float32)


def _proj_call(feat, wt, al, ar):
    return pl.pallas_call(
        _proj_body,
        out_shape=[
            jax.ShapeDtypeStruct((N, HF), jnp.float32),
            jax.ShapeDtypeStruct((N, H), jnp.float32),
            jax.ShapeDtypeStruct((N, H), jnp.float32),
        ],
    )(feat, wt, al, ar)


@functools.partial(
    pl.kernel,
    out_type=jax.ShapeDtypeStruct((NW * N * 4,), jnp.float32),
    mesh=_mesh,
    compiler_params=_sc_params,
    scratch_types=[
        pltpu.VMEM((N * H,), jnp.float32),
        pltpu.VMEM((N * H,), jnp.float32),
        pltpu.VMEM((N * 4,), jnp.float32),
        pltpu.VMEM((EPT,), jnp.int32),
        pltpu.VMEM((EPT,), jnp.int32),
    ],
)
def _stats_kernel(src_hbm, dst_hbm, el_hbm, er_hbm, out_hbm,
                  el_v, er_v, part_v, src_v, dst_v):
    c = lax.axis_index("c")
    s = lax.axis_index("s")
    wid = c * NS + s
    pltpu.sync_copy(el_hbm, el_v)
    pltpu.sync_copy(er_hbm, er_v)
    base_e = wid * EPT
    pltpu.sync_copy(src_hbm.at[pl.ds(base_e, EPT)], src_v)
    pltpu.sync_copy(dst_hbm.at[pl.ds(base_e, EPT)], dst_v)

    zero16 = jnp.zeros((16,), jnp.float32)

    def zbody(i, _):
        part_v[pl.ds(i * 16, 16)] = zero16
        return 0

    lax.fori_loop(0, N * 4 // 16, zbody, 0)

    ones16 = jnp.ones((16,), jnp.float32)

    def vec_body(v, _):
        s16 = src_v[pl.ds(v * 16, 16)]
        d16 = dst_v[pl.ds(v * 16, 16)]
        s3 = s16 * H
        d3 = d16 * H
        d4 = d16 * 4
        for h in range(H):
            elv = plsc.load_gather(el_v, [s3 + h])
            erv = plsc.load_gather(er_v, [d3 + h])
            e = elv + erv
            e = jnp.where(e >= 0.0, e, e * NEG)
            plsc.addupdate_scatter(part_v, [d4 + h], jnp.exp(e))
        plsc.addupdate_scatter(part_v, [d4 + 3], ones16)
        return 0

    lax.fori_loop(0, EPT // 16, vec_body, 0, unroll=2)
    pltpu.sync_copy(part_v, out_hbm.at[pl.ds(wid * N * 4, N * 4)])


def _mid_body(sp_ref, mid_ref):
    stats = jnp.sum(sp_ref[...], axis=0)
    col = lax.rem(lax.broadcasted_iota(jnp.int32, (N * 4,), 0), 4)
    ise = 1.0 / jnp.maximum(stats, 1e-16)
    dn = lax.rsqrt(jnp.maximum(stats, 1.0))
    mid_ref[...] = jnp.where(col < 3, ise, dn)


def _mid_call(spart):
    return pl.pallas_call(
        _mid_body,
        out_shape=jax.ShapeDtypeStruct((N * 4,), jnp.float32),
    )(spart)


@functools.partial(
    pl.kernel,
    out_type=jax.ShapeDtypeStruct((H * E,), jnp.float32),
    mesh=_mesh,
    compiler_params=_sc_params,
    scratch_types=[
        pltpu.VMEM((N * H,), jnp.float32),
        pltpu.VMEM((N * H,), jnp.float32),
        pltpu.VMEM((N * 4,), jnp.float32),
        pltpu.VMEM((CH2,), jnp.int32),
        pltpu.VMEM((CH2,), jnp.int32),
        pltpu.VMEM((H, CH2), jnp.float32),
    ],
)
def _edge_kernel(src_hbm, dst_hbm, el_hbm, er_hbm, mid_hbm, out_hbm,
                 el_v, er_v, mid_v, src_v, dst_v, a_v):
    c = lax.axis_index("c")
    s = lax.axis_index("s")
    wid = c * NS + s
    pltpu.sync_copy(el_hbm, el_v)
    pltpu.sync_copy(er_hbm, er_v)
    pltpu.sync_copy(mid_hbm, mid_v)

    base_e = wid * EPT

    def chunk_body(ci, _):
        off = base_e + ci * CH2
        pltpu.sync_copy(src_hbm.at[pl.ds(off, CH2)], src_v)
        pltpu.sync_copy(dst_hbm.at[pl.ds(off, CH2)], dst_v)

        def vec_body(v, _):
            sl = pl.ds(v * 16, 16)
            s16 = src_v[sl]
            d16 = dst_v[sl]
            s3 = s16 * H
            d3 = d16 * H
            s4 = s16 * 4
            d4 = d16 * 4
            dns = plsc.load_gather(mid_v, [s4 + 3])
            dnd = plsc.load_gather(mid_v, [d4 + 3])
            dd = dns * dnd
            for h in range(H):
                elv = plsc.load_gather(el_v, [s3 + h])
                erv = plsc.load_gather(er_v, [d3 + h])
                e = elv + erv
                e = jnp.where(e >= 0.0, e, e * NEG)
                ise = plsc.load_gather(mid_v, [d4 + h])
                a_v[h, sl] = jnp.exp(e) * ise * dd
            return 0

        lax.fori_loop(0, CH2 // 16, vec_body, 0, unroll=2)
        for h in range(H):
            pltpu.sync_copy(a_v.at[h], out_hbm.at[pl.ds(h * E + off, CH2)])
        return 0

    lax.fori_loop(0, EPT // CH2, chunk_body, 0)


NBUF = 6
PDIST = 3


@functools.partial(
    pl.kernel,
    out_type=jax.ShapeDtypeStruct((NC * H * N, F), jnp.float32),
    mesh=_mesh,
    compiler_params=_sc_params,
    scratch_types=[
        pltpu.VMEM_SHARED((N, F), jnp.float32),
        pltpu.VMEM((EPT,), jnp.int32),
        pltpu.VMEM((NCHUNK, CH), jnp.int32),
        pltpu.VMEM((EPT,), jnp.int32),
        pltpu.VMEM((EPT,), jnp.float32),
        [pltpu.VMEM((CH, F), jnp.float32) for _ in range(NBUF)],
        [pltpu.SemaphoreType.DMA for _ in range(NBUF)],
        [pltpu.SemaphoreType.DMA for _ in range(NBUF)],
    ],
)
def _hop_kernel(src_hbm, dst3_hbm, a_hbm, hin3_hbm, zrow_hbm, out_hbm,
                acc_sh, src_v, dst_v, gi_v, a_v, rows, gsems, ssems):
    c = lax.axis_index("c")
    s = lax.axis_index("s")
    wid = c * NS + s
    base_e = wid * EPT
    pltpu.sync_copy(src_hbm.at[pl.ds(base_e, EPT)], src_v)
    pltpu.sync_copy(dst3_hbm.at[wid], dst_v)
    def fill_and_gather(b, ci, h):
        pltpu.async_copy(hin3_hbm.at[gi_v.at[pl.ds(ci * CH, CH)]], rows[b],
                         gsems[b])

    def scale_rows(b, ci, h):
        rv = rows[b]

        @functools.partial(plsc.parallel_loop, 0, CH, unroll=8)
        def rbody(r):
            a_s = plsc.load_gather(a_v, [jnp.full((16,), ci * CH + r,
                                                  jnp.int32)])
            for q in range(F // 16):
                slc = pl.ds(q * 16, 16)
                rv[r, slc] = rv[r, slc] * a_s

    def process(b, cc, h):
        nxt = cc + PDIST
        bp = (b + PDIST) % NBUF

        @pl.when(nxt < NCHUNK)
        def _():
            @pl.when(cc >= NBUF - PDIST)
            def _():
                pltpu.make_async_copy(rows[bp], acc_sh.at[dst_v.at[0]],
                                      ssems[bp]).wait()
            fill_and_gather(bp, nxt, h)

        pltpu.make_async_copy(hin3_hbm.at[gi_v.at[pl.ds(cc * CH, CH)]],
                              rows[b], gsems[b]).wait()
        scale_rows(b, cc, h)
        pltpu.async_copy(rows[b], acc_sh.at[dst_v.at[cc]], ssems[b], add=True)

    for h in range(H):
        pltpu.sync_copy(zrow_hbm, acc_sh.at[pl.ds(s * RPT, RPT), :])
        pltpu.sync_copy(a_hbm.at[pl.ds(h * E + base_e, EPT)], a_v)

        hn = h * N

        def gfill(k, _):
            sl = pl.ds(k * 16, 16)
            gi_v[sl] = src_v[sl] + hn
            return 0

        lax.fori_loop(0, EPT // 16, gfill, 0, unroll=8)
        plsc.subcore_barrier()

        for b in range(PDIST):
            fill_and_gather(b, b, h)

        def quad_body(p, _):
            ci = p * NBUF
            for b in range(NBUF):
                process(b, ci + b, h)
            return 0

        lax.fori_loop(0, NCHUNK // NBUF, quad_body, 0)
        for b in range(NCHUNK % NBUF):
            process(b, (NCHUNK // NBUF) * NBUF + b, h)

        for b in range(NBUF):
            pltpu.make_async_copy(rows[b], acc_sh.at[dst_v.at[0]],
                                  ssems[b]).wait()
        plsc.subcore_barrier()
        pltpu.sync_copy(acc_sh.at[pl.ds(s * RPT, RPT), :],
                        out_hbm.at[pl.ds(c * H * N + h * N + s * RPT, RPT), :])
        plsc.subcore_barrier()


def _add_body(hp_ref, out3_ref, outf_ref):
    p = hp_ref[0] + hp_ref[1]
    out3_ref[...] = p
    for h in range(H):
        outf_ref[:, h * F:(h + 1) * F] = p[h * N:(h + 1) * N, :]


def _add_call(hp):
    return pl.pallas_call(
        _add_body,
        out_shape=[
            jax.ShapeDtypeStruct((H * N, F), jnp.float32),
            jax.ShapeDtypeStruct((N, HF), jnp.float32),
        ],
    )(hp)


_BN = 2000


def _final_body(ft_ref, h1_ref, h2_ref, h3_ref, pe_ref, hl_ref, hr_ref,
                e3_ref, out_ref):
    pe = pe_ref[...]
    g = [
        ft_ref[...] + pe[0:1, :],
        h1_ref[...] + pe[1:2, :],
        h2_ref[...] + pe[2:3, :],
        h3_ref[...] + pe[3:4, :],
    ]
    hl = hl_ref[...]
    hr = hr_ref[...]
    al_ = jnp.dot(g[0], hl, preferred_element_type=jnp.float32)
    ah = [None] * (KHOP + 1)
    for k in range(KHOP + 1):
        rk = jnp.dot(g[k], hr, preferred_element_type=jnp.float32)
        x = rk + al_
        ah[k] = jnp.where(x >= 0.0, x, x * NEG)
    m = jnp.maximum(jnp.maximum(ah[0], ah[1]), jnp.maximum(ah[2], ah[3]))
    ex = [jnp.exp(a - m) for a in ah]
    ssum = ex[0] + ex[1] + ex[2] + ex[3]
    e3 = e3_ref[...]
    out = jnp.zeros_like(g[0])
    for k in range(KHOP + 1):
        w = ex[k] / ssum
        out = out + g[k] * jnp.dot(w, e3, preferred_element_type=jnp.float32)
    out_ref[...] = out


def _final_call(ft, h1, h2, h3, pe, hl, hr, e3):
    blk = lambda i: (i, 0)
    full2 = pl.BlockSpec((_BN, HF), blk)
    return pl.pallas_call(
        _final_body,
        grid=(N // _BN,),
        in_specs=[
            full2, full2, full2, full2,
            pl.BlockSpec((KHOP + 1, HF), lambda i: (0, 0)),
            pl.BlockSpec((HF, H), lambda i: (0, 0)),
            pl.BlockSpec((HF, H), lambda i: (0, 0)),
            pl.BlockSpec((H, HF), lambda i: (0, 0)),
        ],
        out_specs=full2,
        out_shape=jax.ShapeDtypeStruct((N, HF), jnp.float32),
    )(ft, h1, h2, h3, pe, hl, hr, e3)


def _blockdiag(vec):
    eye = jnp.eye(H, dtype=jnp.float32)
    return (vec[:, :, None] * eye[:, None, :]).reshape(HF, H)


def kernel(feat, edge_index, W_fc, attn_l, attn_r, position_emb,
           hop_attn_l, hop_attn_r):
    src = edge_index[0].astype(jnp.int32)
    dst = edge_index[1].astype(jnp.int32)
    dst3 = dst.reshape(NW, NCHUNK, CH)
    wt = W_fc.T
    al = _blockdiag(attn_l[0])
    ar = _blockdiag(attn_r[0])
    hl = _blockdiag(hop_attn_l[0])
    hr = _blockdiag(hop_attn_r[0])
    e3 = jnp.repeat(jnp.eye(H, dtype=jnp.float32), F, axis=1)
    pe = position_emb.reshape(KHOP + 1, HF)
    zrow = jnp.zeros((RPT, F), jnp.float32)

    ft, el, er = _proj_call(feat, wt, al, ar)
    elf = el.reshape(-1)
    erf = er.reshape(-1)
    spart = _stats_kernel(src, dst, elf, erf).reshape(NW, N * 4)
    mid = _mid_call(spart)
    a4 = _edge_kernel(src, dst, elf, erf, mid)

    ft3 = ft.reshape(N, H, F).transpose(1, 0, 2).reshape(H * N, F)

    h3 = ft3
    hsf = [ft]
    for _ in range(KHOP):
        hp = _hop_kernel(src, dst3, a4, h3, zrow).reshape(NC, H * N, F)
        h3, hflat = _add_call(hp)
        hsf.append(hflat)
    rst = _final_call(hsf[0], hsf[1], hsf[2], hsf[3], pe, hl, hr, e3)
    return rst.reshape(N, H, F)
